# Initial kernel scaffold; baseline (speedup 1.0000x reference)
import jax
import jax.numpy as jnp
from jax.experimental import pallas as pl


def kernel(x, edge_index, W0, b0, W1, b1, bn_gamma, bn_beta, bn_mean, bn_var):
    raise NotImplementedError("write your pallas kernel here")


# speedup vs baseline: 35.7664x; 35.7664x over previous
import functools

import jax
import jax.numpy as jnp
from jax import lax
from jax.experimental import pallas as pl
from jax.experimental.pallas import tpu as pltpu
from jax.experimental.pallas import tpu_sc as plsc

N = 10000
E = 160000
FIN = 256
FHID = 512
FOUT = 256
CW = 128
NCH = 4

NSP = 10240
NSA = 10016
EP = 163840
RB = 1000

_SC_MESH = plsc.VectorSubcoreMesh(core_axis_name="c", subcore_axis_name="s")


@functools.partial(
    pl.kernel,
    mesh=_SC_MESH,
    out_type=jax.ShapeDtypeStruct((2 * NSP,), jnp.float32),
    scratch_types=[
        pltpu.VMEM_SHARED((NSP,), jnp.float32),
        pltpu.VMEM((40, 128), jnp.int32),
        pltpu.VMEM((128,), jnp.float32),
        pltpu.VMEM((640,), jnp.float32),
    ],
)
def _deg_kernel(colp_hbm, out_hbm, shared_deg, col2d, ones_v, zeros_v):
    c = lax.axis_index("c")
    s = lax.axis_index("s")
    wid = c * 16 + s
    for k in range(8):
        ones_v[pl.ds(k * 16, 16)] = jnp.ones((16,), jnp.float32)
    for k in range(40):
        zeros_v[pl.ds(k * 16, 16)] = jnp.zeros((16,), jnp.float32)
    pltpu.sync_copy(zeros_v, shared_deg.at[pl.ds(s * 640, 640)])
    base = wid * 5120
    for j in range(40):
        pltpu.sync_copy(colp_hbm.at[pl.ds(base + j * 128, 128)], col2d.at[j])
    plsc.subcore_barrier()
    for j in range(40):
        pltpu.sync_copy(ones_v, shared_deg.at[col2d.at[j]], add=True)
    plsc.subcore_barrier()
    pltpu.sync_copy(shared_deg.at[pl.ds(s * 640, 640)],
                    out_hbm.at[pl.ds(c * NSP + s * 640, 640)])


@functools.partial(
    pl.kernel,
    mesh=_SC_MESH,
    out_type=jax.ShapeDtypeStruct((NCH, N, CW), jnp.float32),
    scratch_types=[
        pltpu.VMEM_SHARED((NSA, CW), jnp.float32),
        pltpu.VMEM((2, 512), jnp.int32),
        pltpu.VMEM((80, 128), jnp.int32),
        pltpu.VMEM((2, 128, CW), jnp.float32),
        pltpu.SemaphoreType.DMA((2,)),
        pltpu.SemaphoreType.DMA((2,)),
        pltpu.SemaphoreType.DMA((2,)),
    ],
)
def _prop_kernel(g_hbm, rowadj_hbm, colp_hbm, s_hbm,
                 shared, rowb, col2d, dbuf, rsem, gsem, ssem):
    c = lax.axis_index("c")
    s = lax.axis_index("s")
    ebase = s * 10240
    for j in range(80):
        pltpu.sync_copy(colp_hbm.at[pl.ds(ebase + j * 128, 128)], col2d.at[j])
    for p in range(2):
        chunk = c * 2 + p
        rbase = chunk * EP + ebase
        pltpu.sync_copy(g_hbm.at[pl.ds(chunk * N + s * 624, 624)],
                        shared.at[pl.ds(s * 624, 624)])

        @pl.when(s == 15)
        def _():
            pltpu.sync_copy(g_hbm.at[pl.ds(chunk * N + 9984, 16)],
                            shared.at[pl.ds(9984, 16)])
        plsc.subcore_barrier()

        pltpu.async_copy(rowadj_hbm.at[pl.ds(rbase, 512)],
                         rowb.at[0], rsem.at[0])

        def _idx_desc(grp, slot):
            return pltpu.make_async_copy(
                rowadj_hbm.at[pl.ds(rbase + grp * 512, 512)],
                rowb.at[slot], rsem.at[slot])

        def _gather(j, b):
            jo = j // 4
            off = pl.multiple_of((j % 4) * 128, 128)
            return pltpu.make_async_copy(
                g_hbm.at[rowb.at[jo % 2, pl.ds(off, 128)]],
                dbuf.at[b], gsem.at[b])

        def _scatter(j, b):
            return pltpu.make_async_copy(
                dbuf.at[b], shared.at[col2d.at[j]], ssem.at[b])

        def _step(j, _):
            b = j % 2
            jo = j // 4
            k = j % 4

            @pl.when(j >= 1)
            def _():
                _gather(j - 1, 1 - b).wait()

                @pl.when(j >= 2)
                def _():
                    _scatter(j - 2, b).wait()
                pltpu.async_copy(dbuf.at[1 - b], shared.at[col2d.at[j - 1]],
                                 ssem.at[1 - b], add=True)

            @pl.when(k == 0)
            def _():
                _idx_desc(jo, jo % 2).wait()

                @pl.when(jo + 1 < 20)
                def _():
                    _idx_desc(jo + 1, (jo + 1) % 2).start()

            _gather(j, b).start()
            return 0

        lax.fori_loop(0, 80, _step, 0)
        _gather(79, 1).wait()
        _scatter(78, 0).wait()
        pltpu.async_copy(dbuf.at[1], shared.at[col2d.at[79]],
                         ssem.at[1], add=True)
        _scatter(79, 1).wait()
        plsc.subcore_barrier()
        pltpu.sync_copy(shared.at[pl.ds(s * 624, 624)],
                        s_hbm.at[chunk, pl.ds(s * 624, 624)])

        @pl.when(s == 15)
        def _():
            pltpu.sync_copy(shared.at[pl.ds(9984, 16)],
                            s_hbm.at[chunk, pl.ds(9984, 16)])
        if p == 0:
            plsc.subcore_barrier()


def _k1_body(x_ref, w0_ref, b0_ref, p0_ref, p1_ref, g_ref):
    h = jnp.dot(x_ref[...], w0_ref[...], preferred_element_type=jnp.float32)
    h = jnp.maximum(h + b0_ref[...], 0.0)
    dinv = lax.rsqrt(p0_ref[...] + p1_ref[...] + 1.0)
    g = h * dinv
    for cc in range(NCH):
        g_ref[cc] = g[:, cc * CW:(cc + 1) * CW]


def _k1_call(x, W0, b0r, p0, p1):
    return pl.pallas_call(
        _k1_body,
        grid=(N // RB,),
        in_specs=[
            pl.BlockSpec((RB, FIN), lambda i: (i, 0)),
            pl.BlockSpec((FIN, FHID), lambda i: (0, 0)),
            pl.BlockSpec((1, FHID), lambda i: (0, 0)),
            pl.BlockSpec((RB, 1), lambda i: (i, 0)),
            pl.BlockSpec((RB, 1), lambda i: (i, 0)),
        ],
        out_specs=pl.BlockSpec((NCH, RB, CW), lambda i: (0, i, 0)),
        out_shape=jax.ShapeDtypeStruct((NCH, N, CW), jnp.float32),
        compiler_params=pltpu.CompilerParams(
            dimension_semantics=("arbitrary",)),
    )(x, W0, b0r, p0, p1)


def _k2_body(s_ref, p0_ref, p1_ref, w1_ref, b1_ref, gam_ref, bet_ref,
             mu_ref, var_ref, o_ref):
    dinv = lax.rsqrt(p0_ref[...] + p1_ref[...] + 1.0)
    ii = lax.broadcasted_iota(jnp.int32, (CW, CW), 0)
    jj = lax.broadcasted_iota(jnp.int32, (CW, CW), 1)
    a_op = (jnp.where(ii == jj, 2.0, 0.0)
            + jnp.where((ii // 2) == (jj // 2), 0.5, 0.0)
            + jnp.where((ii // 4) == (jj // 4), 0.25, 0.0))
    acc = jnp.zeros((RB, FOUT), jnp.float32)
    for cc in range(NCH):
        t = s_ref[cc] * dinv
        wc = jnp.dot(a_op, w1_ref[cc], preferred_element_type=jnp.float32)
        acc = acc + jnp.dot(t, wc, preferred_element_type=jnp.float32)
    scale = gam_ref[...] * lax.rsqrt(var_ref[...] + 1e-5)
    o_ref[...] = acc * scale + (b1_ref[...] - mu_ref[...]) * scale + bet_ref[...]


def _k2_call(S, p0, p1, W1r, b1r, gamr, betr, mur, varr):
    vec = pl.BlockSpec((1, FOUT), lambda i: (0, 0))
    return pl.pallas_call(
        _k2_body,
        grid=(N // RB,),
        in_specs=[
            pl.BlockSpec((NCH, RB, CW), lambda i: (0, i, 0)),
            pl.BlockSpec((RB, 1), lambda i: (i, 0)),
            pl.BlockSpec((RB, 1), lambda i: (i, 0)),
            pl.BlockSpec((NCH, CW, FOUT), lambda i: (0, 0, 0)),
            vec, vec, vec, vec, vec,
        ],
        out_specs=pl.BlockSpec((RB, FOUT), lambda i: (i, 0)),
        out_shape=jax.ShapeDtypeStruct((N, FOUT), jnp.float32),
        compiler_params=pltpu.CompilerParams(
            dimension_semantics=("arbitrary",)),
    )(S, p0, p1, W1r, b1r, gamr, betr, mur, varr)


def kernel(x, edge_index, W0, b0, W1, b1, bn_gamma, bn_beta, bn_mean, bn_var):
    row = edge_index[0]
    col = edge_index[1]
    npad = EP - E
    pad_r = jnp.arange(npad, dtype=jnp.int32) % N
    pad_c = N + jnp.arange(npad, dtype=jnp.int32) % (NSA - N)
    rowp = jnp.concatenate([row, pad_r])
    colp = jnp.concatenate([col, pad_c])
    row_adj = (rowp[None, :]
               + (jnp.arange(NCH, dtype=jnp.int32)[:, None] * N)).reshape(-1)

    degp = _deg_kernel(colp)
    p0 = degp[:N].reshape(N, 1)
    p1 = degp[NSP:NSP + N].reshape(N, 1)

    g4 = _k1_call(x, W0, b0.reshape(1, FHID), p0, p1)
    g_flat = g4.reshape(NCH * N, CW)

    S = _prop_kernel(g_flat, row_adj, colp)

    return _k2_call(S, p0, p1, W1.reshape(NCH, CW, FOUT),
                    b1.reshape(1, FOUT), bn_gamma.reshape(1, FOUT),
                    bn_beta.reshape(1, FOUT), bn_mean.reshape(1, FOUT),
                    bn_var.reshape(1, FOUT))


# speedup vs baseline: 41.2597x; 1.1536x over previous
import functools

import jax
import jax.numpy as jnp
from jax import lax
from jax.experimental import pallas as pl
from jax.experimental.pallas import tpu as pltpu
from jax.experimental.pallas import tpu_sc as plsc

N = 10000
E = 160000
FIN = 256
FHID = 512
FOUT = 256
CW = 128
NCH = 4

NSP = 10240
NSA = 10008
EP = 163840
RB = 1000

_SC_MESH = plsc.VectorSubcoreMesh(core_axis_name="c", subcore_axis_name="s")


@functools.partial(
    pl.kernel,
    mesh=_SC_MESH,
    out_type=jax.ShapeDtypeStruct((2 * NSP,), jnp.float32),
    scratch_types=[
        pltpu.VMEM_SHARED((NSP,), jnp.float32),
        pltpu.VMEM((40, 128), jnp.int32),
        pltpu.VMEM((128,), jnp.float32),
        pltpu.VMEM((640,), jnp.float32),
        pltpu.SemaphoreType.DMA,
        pltpu.SemaphoreType.DMA((4,)),
    ],
)
def _deg_kernel(colp_hbm, out_hbm, shared_deg, col2d, ones_v, zeros_v,
                lsem, dsem):
    c = lax.axis_index("c")
    s = lax.axis_index("s")
    wid = c * 16 + s
    for k in range(8):
        ones_v[pl.ds(k * 16, 16)] = jnp.ones((16,), jnp.float32)
    for k in range(40):
        zeros_v[pl.ds(k * 16, 16)] = jnp.zeros((16,), jnp.float32)
    pltpu.sync_copy(zeros_v, shared_deg.at[pl.ds(s * 640, 640)])
    base = wid * 5120
    for j in range(40):
        pltpu.async_copy(colp_hbm.at[pl.ds(base + j * 128, 128)],
                         col2d.at[j], lsem)
    for j in range(40):
        pltpu.make_async_copy(colp_hbm.at[pl.ds(base + j * 128, 128)],
                              col2d.at[j], lsem).wait()
    plsc.subcore_barrier()
    for j in range(40):
        if j >= 4:
            pltpu.make_async_copy(ones_v, shared_deg.at[col2d.at[j - 4]],
                                  dsem.at[(j - 4) % 4]).wait()
        pltpu.async_copy(ones_v, shared_deg.at[col2d.at[j]],
                         dsem.at[j % 4], add=True)
    for j in range(36, 40):
        pltpu.make_async_copy(ones_v, shared_deg.at[col2d.at[j]],
                              dsem.at[j % 4]).wait()
    plsc.subcore_barrier()
    pltpu.sync_copy(shared_deg.at[pl.ds(s * 640, 640)],
                    out_hbm.at[pl.ds(c * NSP + s * 640, 640)])


@functools.partial(
    pl.kernel,
    mesh=_SC_MESH,
    out_type=jax.ShapeDtypeStruct((NCH, N, CW), jnp.float32),
    scratch_types=[
        pltpu.VMEM_SHARED((NSA, CW), jnp.float32),
        pltpu.VMEM((3, 256), jnp.int32),
        pltpu.VMEM((3, 2, 128), jnp.int32),
        pltpu.VMEM((3, 128, CW), jnp.float32),
        pltpu.SemaphoreType.DMA((3,)),
        pltpu.SemaphoreType.DMA((3,)),
        pltpu.SemaphoreType.DMA((3,)),
        pltpu.SemaphoreType.DMA((3,)),
    ],
)
def _prop_kernel(g_hbm, rowadj_hbm, colp_hbm, s_hbm,
                 shared, rowb, colb, dbuf, rsem, csem, gsem, ssem):
    c = lax.axis_index("c")
    s = lax.axis_index("s")
    ebase = s * 10240
    for p in range(2):
        chunk = c * 2 + p
        rbase = chunk * EP + ebase
        pltpu.sync_copy(g_hbm.at[pl.ds(chunk * N + s * 624, 624)],
                        shared.at[pl.ds(s * 624, 624)])

        @pl.when(s == 15)
        def _():
            pltpu.sync_copy(g_hbm.at[pl.ds(chunk * N + 9984, 16)],
                            shared.at[pl.ds(9984, 16)])
        plsc.subcore_barrier()

        def _ridx(grp):
            return pltpu.make_async_copy(
                rowadj_hbm.at[pl.ds(rbase + grp * 256, 256)],
                rowb.at[grp % 3], rsem.at[grp % 3])

        def _cidx(grp, half):
            return pltpu.make_async_copy(
                colp_hbm.at[pl.ds(ebase + grp * 256 + half * 128, 128)],
                colb.at[grp % 3, half], csem.at[grp % 3])

        def _gather(j):
            jo = j // 2
            off = pl.multiple_of((j % 2) * 128, 128)
            return pltpu.make_async_copy(
                g_hbm.at[rowb.at[jo % 3, pl.ds(off, 128)]],
                dbuf.at[j % 3], gsem.at[j % 3])

        def _scatter(j):
            return pltpu.make_async_copy(
                dbuf.at[j % 3], shared.at[colb.at[(j // 2) % 3, j % 2]],
                ssem.at[j % 3])

        _ridx(0).start()
        _cidx(0, 0).start()
        _cidx(0, 1).start()

        def _step(j, _):
            jo = j // 2
            k = j % 2

            @pl.when(j >= 1)
            def _():
                _gather(j - 1).wait()
                pltpu.async_copy(
                    dbuf.at[(j - 1) % 3],
                    shared.at[colb.at[((j - 1) // 2) % 3, (j - 1) % 2]],
                    ssem.at[(j - 1) % 3], add=True)

            @pl.when(j >= 3)
            def _():
                _scatter(j - 3).wait()

            @pl.when(k == 0)
            def _():
                _ridx(jo).wait()
                _cidx(jo, 0).wait()
                _cidx(jo, 1).wait()

                @pl.when(jo + 1 < 40)
                def _():
                    _ridx(jo + 1).start()
                    _cidx(jo + 1, 0).start()
                    _cidx(jo + 1, 1).start()

            _gather(j).start()
            return 0

        lax.fori_loop(0, 80, _step, 0)
        _gather(79).wait()
        pltpu.async_copy(dbuf.at[79 % 3],
                         shared.at[colb.at[(79 // 2) % 3, 1]],
                         ssem.at[79 % 3], add=True)
        _scatter(77).wait()
        _scatter(78).wait()
        _scatter(79).wait()
        plsc.subcore_barrier()
        pltpu.sync_copy(shared.at[pl.ds(s * 624, 624)],
                        s_hbm.at[chunk, pl.ds(s * 624, 624)])

        @pl.when(s == 15)
        def _():
            pltpu.sync_copy(shared.at[pl.ds(9984, 16)],
                            s_hbm.at[chunk, pl.ds(9984, 16)])
        if p == 0:
            plsc.subcore_barrier()


def _k1_body(x_ref, w0_ref, b0_ref, p0_ref, p1_ref, g_ref):
    h = jnp.dot(x_ref[...], w0_ref[...], preferred_element_type=jnp.float32)
    h = jnp.maximum(h + b0_ref[...], 0.0)
    dinv = lax.rsqrt(p0_ref[...] + p1_ref[...] + 1.0)
    g = h * dinv
    for cc in range(NCH):
        g_ref[cc] = g[:, cc * CW:(cc + 1) * CW]


def _k1_call(x, W0, b0r, p0, p1):
    return pl.pallas_call(
        _k1_body,
        grid=(N // RB,),
        in_specs=[
            pl.BlockSpec((RB, FIN), lambda i: (i, 0)),
            pl.BlockSpec((FIN, FHID), lambda i: (0, 0)),
            pl.BlockSpec((1, FHID), lambda i: (0, 0)),
            pl.BlockSpec((RB, 1), lambda i: (i, 0)),
            pl.BlockSpec((RB, 1), lambda i: (i, 0)),
        ],
        out_specs=pl.BlockSpec((NCH, RB, CW), lambda i: (0, i, 0)),
        out_shape=jax.ShapeDtypeStruct((NCH, N, CW), jnp.float32),
        compiler_params=pltpu.CompilerParams(
            dimension_semantics=("arbitrary",)),
    )(x, W0, b0r, p0, p1)


def _k2_body(s_ref, p0_ref, p1_ref, w1_ref, b1_ref, gam_ref, bet_ref,
             mu_ref, var_ref, o_ref):
    dinv = lax.rsqrt(p0_ref[...] + p1_ref[...] + 1.0)
    ii = lax.broadcasted_iota(jnp.int32, (CW, CW), 0)
    jj = lax.broadcasted_iota(jnp.int32, (CW, CW), 1)
    a_op = (jnp.where(ii == jj, 2.0, 0.0)
            + jnp.where((ii // 2) == (jj // 2), 0.5, 0.0)
            + jnp.where((ii // 4) == (jj // 4), 0.25, 0.0))
    acc = jnp.zeros((RB, FOUT), jnp.float32)
    for cc in range(NCH):
        t = s_ref[cc] * dinv
        wc = jnp.dot(a_op, w1_ref[cc], preferred_element_type=jnp.float32)
        acc = acc + jnp.dot(t, wc, preferred_element_type=jnp.float32)
    scale = gam_ref[...] * lax.rsqrt(var_ref[...] + 1e-5)
    o_ref[...] = acc * scale + (b1_ref[...] - mu_ref[...]) * scale + bet_ref[...]


def _k2_call(S, p0, p1, W1r, b1r, gamr, betr, mur, varr):
    vec = pl.BlockSpec((1, FOUT), lambda i: (0, 0))
    return pl.pallas_call(
        _k2_body,
        grid=(N // RB,),
        in_specs=[
            pl.BlockSpec((NCH, RB, CW), lambda i: (0, i, 0)),
            pl.BlockSpec((RB, 1), lambda i: (i, 0)),
            pl.BlockSpec((RB, 1), lambda i: (i, 0)),
            pl.BlockSpec((NCH, CW, FOUT), lambda i: (0, 0, 0)),
            vec, vec, vec, vec, vec,
        ],
        out_specs=pl.BlockSpec((RB, FOUT), lambda i: (i, 0)),
        out_shape=jax.ShapeDtypeStruct((N, FOUT), jnp.float32),
        compiler_params=pltpu.CompilerParams(
            dimension_semantics=("arbitrary",)),
    )(S, p0, p1, W1r, b1r, gamr, betr, mur, varr)


def kernel(x, edge_index, W0, b0, W1, b1, bn_gamma, bn_beta, bn_mean, bn_var):
    row = edge_index[0]
    col = edge_index[1]
    npad = EP - E
    pad_r = jnp.arange(npad, dtype=jnp.int32) % N
    pad_c = N + jnp.arange(npad, dtype=jnp.int32) % (NSA - N)
    rowp = jnp.concatenate([row, pad_r])
    colp = jnp.concatenate([col, pad_c])
    row_adj = (rowp[None, :]
               + (jnp.arange(NCH, dtype=jnp.int32)[:, None] * N)).reshape(-1)

    degp = _deg_kernel(colp)
    p0 = degp[:N].reshape(N, 1)
    p1 = degp[NSP:NSP + N].reshape(N, 1)

    g4 = _k1_call(x, W0, b0.reshape(1, FHID), p0, p1)
    g_flat = g4.reshape(NCH * N, CW)

    S = _prop_kernel(g_flat, row_adj, colp)

    return _k2_call(S, p0, p1, W1.reshape(NCH, CW, FOUT),
                    b1.reshape(1, FOUT), bn_gamma.reshape(1, FOUT),
                    bn_beta.reshape(1, FOUT), bn_mean.reshape(1, FOUT),
                    bn_var.reshape(1, FOUT))


# speedup vs baseline: 50.1105x; 1.2145x over previous
import functools

import jax
import jax.numpy as jnp
from jax import lax
from jax.experimental import pallas as pl
from jax.experimental.pallas import tpu as pltpu
from jax.experimental.pallas import tpu_sc as plsc

N = 10000
E = 160000
FIN = 256
FHID = 512
FOUT = 256
CW = 128
NCH = 4

NSP = 10240
NSA = 10008
EP = 163840
RB = 2000

_SC_MESH = plsc.VectorSubcoreMesh(core_axis_name="c", subcore_axis_name="s")


@functools.partial(
    pl.kernel,
    mesh=_SC_MESH,
    out_type=jax.ShapeDtypeStruct((2 * NSP,), jnp.float32),
    scratch_types=[
        pltpu.VMEM_SHARED((NSP,), jnp.float32),
        pltpu.VMEM((40, 128), jnp.int32),
        pltpu.VMEM((128,), jnp.float32),
        pltpu.VMEM((640,), jnp.float32),
        pltpu.SemaphoreType.DMA,
        pltpu.SemaphoreType.DMA((4,)),
    ],
)
def _deg_kernel(colp_hbm, out_hbm, shared_deg, col2d, ones_v, zeros_v,
                lsem, dsem):
    c = lax.axis_index("c")
    s = lax.axis_index("s")
    wid = c * 16 + s
    for k in range(8):
        ones_v[pl.ds(k * 16, 16)] = jnp.ones((16,), jnp.float32)
    for k in range(40):
        zeros_v[pl.ds(k * 16, 16)] = jnp.zeros((16,), jnp.float32)
    pltpu.sync_copy(zeros_v, shared_deg.at[pl.ds(s * 640, 640)])
    base = wid * 5120
    for j in range(40):
        pltpu.async_copy(colp_hbm.at[pl.ds(base + j * 128, 128)],
                         col2d.at[j], lsem)
    for j in range(40):
        pltpu.make_async_copy(colp_hbm.at[pl.ds(base + j * 128, 128)],
                              col2d.at[j], lsem).wait()
    plsc.subcore_barrier()
    for j in range(40):
        if j >= 4:
            pltpu.make_async_copy(ones_v, shared_deg.at[col2d.at[j - 4]],
                                  dsem.at[(j - 4) % 4]).wait()
        pltpu.async_copy(ones_v, shared_deg.at[col2d.at[j]],
                         dsem.at[j % 4], add=True)
    for j in range(36, 40):
        pltpu.make_async_copy(ones_v, shared_deg.at[col2d.at[j]],
                              dsem.at[j % 4]).wait()
    plsc.subcore_barrier()
    pltpu.sync_copy(shared_deg.at[pl.ds(s * 640, 640)],
                    out_hbm.at[pl.ds(c * NSP + s * 640, 640)])


@functools.partial(
    pl.kernel,
    mesh=_SC_MESH,
    out_type=jax.ShapeDtypeStruct((NCH, N, CW), jnp.float32),
    scratch_types=[
        pltpu.VMEM_SHARED((NSA, CW), jnp.float32),
        pltpu.VMEM((3, 256), jnp.int32),
        pltpu.VMEM((3, 2, 128), jnp.int32),
        pltpu.VMEM((3, 128, CW), jnp.float32),
        pltpu.SemaphoreType.DMA((3,)),
        pltpu.SemaphoreType.DMA((3,)),
        pltpu.SemaphoreType.DMA((3,)),
        pltpu.SemaphoreType.DMA((3,)),
    ],
)
def _prop_kernel(g_hbm, rowp_hbm, colp_hbm, s_hbm,
                 shared, rowb, colb, dbuf, rsem, csem, gsem, ssem):
    c = lax.axis_index("c")
    s = lax.axis_index("s")
    ebase = s * 10240
    for p in range(2):
        chunk = c * 2 + p
        pltpu.sync_copy(g_hbm.at[pl.ds(chunk * N + s * 624, 624)],
                        shared.at[pl.ds(s * 624, 624)])

        @pl.when(s == 15)
        def _():
            pltpu.sync_copy(g_hbm.at[pl.ds(chunk * N + 9984, 16)],
                            shared.at[pl.ds(9984, 16)])
        plsc.subcore_barrier()

        def _ridx(grp):
            return pltpu.make_async_copy(
                rowp_hbm.at[pl.ds(ebase + grp * 256, 256)],
                rowb.at[grp % 3], rsem.at[grp % 3])

        def _idx_ready(gi):
            _ridx(gi).wait()
            _cidx(gi, 0).wait()
            _cidx(gi, 1).wait()
            for t in range(16):
                rowb[gi % 3, pl.ds(t * 16, 16)] = (
                    rowb[gi % 3, pl.ds(t * 16, 16)] + chunk * N)

        def _cidx(grp, half):
            return pltpu.make_async_copy(
                colp_hbm.at[pl.ds(ebase + grp * 256 + half * 128, 128)],
                colb.at[grp % 3, half], csem.at[grp % 3])

        def _gather(j):
            jo = j // 2
            off = pl.multiple_of((j % 2) * 128, 128)
            return pltpu.make_async_copy(
                g_hbm.at[rowb.at[jo % 3, pl.ds(off, 128)]],
                dbuf.at[j % 3], gsem.at[j % 3])

        def _scatter(j):
            return pltpu.make_async_copy(
                dbuf.at[j % 3], shared.at[colb.at[(j // 2) % 3, j % 2]],
                ssem.at[j % 3])

        _ridx(0).start()
        _cidx(0, 0).start()
        _cidx(0, 1).start()
        _idx_ready(0)
        _ridx(1).start()
        _cidx(1, 0).start()
        _cidx(1, 1).start()
        _gather(0).start()
        _gather(1).start()

        def _step(j, _):
            _gather(j).wait()
            pltpu.async_copy(
                dbuf.at[j % 3],
                shared.at[colb.at[(j // 2) % 3, j % 2]],
                ssem.at[j % 3], add=True)

            @pl.when(j >= 1)
            def _():
                _scatter(j - 1).wait()

            @pl.when(j + 2 < 80)
            def _():
                @pl.when((j + 2) % 2 == 0)
                def _():
                    gi = (j + 2) // 2
                    _idx_ready(gi)

                    @pl.when(gi + 1 < 40)
                    def _():
                        _ridx(gi + 1).start()
                        _cidx(gi + 1, 0).start()
                        _cidx(gi + 1, 1).start()
                _gather(j + 2).start()
            return 0

        lax.fori_loop(0, 80, _step, 0)
        _scatter(79).wait()
        plsc.subcore_barrier()
        pltpu.sync_copy(shared.at[pl.ds(s * 624, 624)],
                        s_hbm.at[chunk, pl.ds(s * 624, 624)])

        @pl.when(s == 15)
        def _():
            pltpu.sync_copy(shared.at[pl.ds(9984, 16)],
                            s_hbm.at[chunk, pl.ds(9984, 16)])
        if p == 0:
            plsc.subcore_barrier()


def _k1_body(x_ref, w0_ref, b0_ref, p0_ref, p1_ref, g_ref):
    h = jnp.dot(x_ref[...], w0_ref[...], preferred_element_type=jnp.float32)
    h = jnp.maximum(h + b0_ref[...], 0.0)
    dinv = lax.rsqrt(p0_ref[...] + p1_ref[...] + 1.0)
    g = h * dinv
    for cc in range(NCH):
        g_ref[cc] = g[:, cc * CW:(cc + 1) * CW]


def _k1_call(x, W0, b0r, p0, p1):
    return pl.pallas_call(
        _k1_body,
        grid=(N // RB,),
        in_specs=[
            pl.BlockSpec((RB, FIN), lambda i: (i, 0)),
            pl.BlockSpec((FIN, FHID), lambda i: (0, 0)),
            pl.BlockSpec((1, FHID), lambda i: (0, 0)),
            pl.BlockSpec((RB, 1), lambda i: (i, 0)),
            pl.BlockSpec((RB, 1), lambda i: (i, 0)),
        ],
        out_specs=pl.BlockSpec((NCH, RB, CW), lambda i: (0, i, 0)),
        out_shape=jax.ShapeDtypeStruct((NCH, N, CW), jnp.float32),
        compiler_params=pltpu.CompilerParams(
            dimension_semantics=("arbitrary",)),
    )(x, W0, b0r, p0, p1)


def _k2_body(s_ref, p0_ref, p1_ref, w1_ref, b1_ref, gam_ref, bet_ref,
             mu_ref, var_ref, o_ref):
    dinv = lax.rsqrt(p0_ref[...] + p1_ref[...] + 1.0)
    ii = lax.broadcasted_iota(jnp.int32, (CW, CW), 0)
    jj = lax.broadcasted_iota(jnp.int32, (CW, CW), 1)
    a_op = (jnp.where(ii == jj, 2.0, 0.0)
            + jnp.where((ii // 2) == (jj // 2), 0.5, 0.0)
            + jnp.where((ii // 4) == (jj // 4), 0.25, 0.0))
    acc = jnp.zeros((RB, FOUT), jnp.float32)
    for cc in range(NCH):
        t = s_ref[cc] * dinv
        wc = jnp.dot(a_op, w1_ref[cc], preferred_element_type=jnp.float32)
        acc = acc + jnp.dot(t, wc, preferred_element_type=jnp.float32)
    scale = gam_ref[...] * lax.rsqrt(var_ref[...] + 1e-5)
    o_ref[...] = acc * scale + (b1_ref[...] - mu_ref[...]) * scale + bet_ref[...]


def _k2_call(S, p0, p1, W1r, b1r, gamr, betr, mur, varr):
    vec = pl.BlockSpec((1, FOUT), lambda i: (0, 0))
    return pl.pallas_call(
        _k2_body,
        grid=(N // RB,),
        in_specs=[
            pl.BlockSpec((NCH, RB, CW), lambda i: (0, i, 0)),
            pl.BlockSpec((RB, 1), lambda i: (i, 0)),
            pl.BlockSpec((RB, 1), lambda i: (i, 0)),
            pl.BlockSpec((NCH, CW, FOUT), lambda i: (0, 0, 0)),
            vec, vec, vec, vec, vec,
        ],
        out_specs=pl.BlockSpec((RB, FOUT), lambda i: (i, 0)),
        out_shape=jax.ShapeDtypeStruct((N, FOUT), jnp.float32),
        compiler_params=pltpu.CompilerParams(
            dimension_semantics=("arbitrary",)),
    )(S, p0, p1, W1r, b1r, gamr, betr, mur, varr)


def kernel(x, edge_index, W0, b0, W1, b1, bn_gamma, bn_beta, bn_mean, bn_var):
    row = edge_index[0]
    col = edge_index[1]
    npad = EP - E
    pad_r = jnp.arange(npad, dtype=jnp.int32) % N
    pad_c = N + jnp.arange(npad, dtype=jnp.int32) % (NSA - N)
    rowp = jnp.concatenate([row, pad_r])
    colp = jnp.concatenate([col, pad_c])

    degp = _deg_kernel(colp)
    p0 = degp[:N].reshape(N, 1)
    p1 = degp[NSP:NSP + N].reshape(N, 1)

    g4 = _k1_call(x, W0, b0.reshape(1, FHID), p0, p1)
    g_flat = g4.reshape(NCH * N, CW)

    S = _prop_kernel(g_flat, rowp, colp)

    return _k2_call(S, p0, p1, W1.reshape(NCH, CW, FOUT),
                    b1.reshape(1, FOUT), bn_gamma.reshape(1, FOUT),
                    bn_beta.reshape(1, FOUT), bn_mean.reshape(1, FOUT),
                    bn_var.reshape(1, FOUT))


# speedup vs baseline: 52.1241x; 1.0402x over previous
import functools

import jax
import jax.numpy as jnp
from jax import lax
from jax.experimental import pallas as pl
from jax.experimental.pallas import tpu as pltpu
from jax.experimental.pallas import tpu_sc as plsc

N = 10000
E = 160000
FIN = 256
FHID = 512
FOUT = 256
CW = 128
NCH = 4

NSP = 10240
NSA = 10000
RB = 2000

_SC_MESH = plsc.VectorSubcoreMesh(core_axis_name="c", subcore_axis_name="s")


@functools.partial(
    pl.kernel,
    mesh=_SC_MESH,
    out_type=jax.ShapeDtypeStruct((2 * NSP,), jnp.float32),
    scratch_types=[
        pltpu.VMEM_SHARED((NSP,), jnp.float32),
        pltpu.VMEM((40, 128), jnp.int32),
        pltpu.VMEM((128,), jnp.float32),
        pltpu.VMEM((640,), jnp.float32),
        pltpu.SemaphoreType.DMA,
        pltpu.SemaphoreType.DMA((4,)),
    ],
)
def _deg_kernel(colp_hbm, out_hbm, shared_deg, col2d, ones_v, zeros_v,
                lsem, dsem):
    c = lax.axis_index("c")
    s = lax.axis_index("s")
    wid = c * 16 + s
    nst = jnp.where(wid < 31, 40, 10)
    for k in range(8):
        ones_v[pl.ds(k * 16, 16)] = jnp.ones((16,), jnp.float32)
    for k in range(40):
        zeros_v[pl.ds(k * 16, 16)] = jnp.zeros((16,), jnp.float32)
    pltpu.sync_copy(zeros_v, shared_deg.at[pl.ds(s * 640, 640)])
    base = wid * 5120
    for j in range(40):
        @pl.when(j < nst)
        def _():
            pltpu.async_copy(colp_hbm.at[pl.ds(base + j * 128, 128)],
                             col2d.at[j], lsem)
    for j in range(40):
        @pl.when(j < nst)
        def _():
            pltpu.make_async_copy(colp_hbm.at[pl.ds(base + j * 128, 128)],
                                  col2d.at[j], lsem).wait()
    plsc.subcore_barrier()
    for j in range(40):
        if j >= 4:
            @pl.when(j - 4 < nst)
            def _():
                pltpu.make_async_copy(ones_v, shared_deg.at[col2d.at[j - 4]],
                                      dsem.at[(j - 4) % 4]).wait()

        @pl.when(j < nst)
        def _():
            pltpu.async_copy(ones_v, shared_deg.at[col2d.at[j]],
                             dsem.at[j % 4], add=True)
    for j in range(36, 40):
        @pl.when(j < nst)
        def _():
            pltpu.make_async_copy(ones_v, shared_deg.at[col2d.at[j]],
                                  dsem.at[j % 4]).wait()
    plsc.subcore_barrier()
    pltpu.sync_copy(shared_deg.at[pl.ds(s * 640, 640)],
                    out_hbm.at[pl.ds(c * NSP + s * 640, 640)])


@functools.partial(
    pl.kernel,
    mesh=_SC_MESH,
    out_type=jax.ShapeDtypeStruct((NCH, N, CW), jnp.float32),
    scratch_types=[
        pltpu.VMEM_SHARED((NSA, CW), jnp.float32),
        pltpu.VMEM((3, 256), jnp.int32),
        pltpu.VMEM((3, 2, 128), jnp.int32),
        pltpu.VMEM((3, 128, CW), jnp.float32),
        pltpu.SemaphoreType.DMA((3,)),
        pltpu.SemaphoreType.DMA((3,)),
        pltpu.SemaphoreType.DMA((3,)),
        pltpu.SemaphoreType.DMA((3,)),
    ],
)
def _prop_kernel(g_hbm, rowp_hbm, colp_hbm, s_hbm,
                 shared, rowb, colb, dbuf, rsem, csem, gsem, ssem):
    c = lax.axis_index("c")
    s = lax.axis_index("s")
    ebase = s * 10240
    nst = jnp.where(s == 15, 50, 80)
    ngrp = jnp.where(s == 15, 25, 40)
    for p in range(2):
        chunk = c * 2 + p
        pltpu.sync_copy(g_hbm.at[pl.ds(chunk * N + s * 624, 624)],
                        shared.at[pl.ds(s * 624, 624)])

        @pl.when(s == 15)
        def _():
            pltpu.sync_copy(g_hbm.at[pl.ds(chunk * N + 9984, 16)],
                            shared.at[pl.ds(9984, 16)])
        plsc.subcore_barrier()

        def _ridx(grp):
            return pltpu.make_async_copy(
                rowp_hbm.at[pl.ds(ebase + grp * 256, 256)],
                rowb.at[grp % 3], rsem.at[grp % 3])

        def _idx_ready(gi):
            _ridx(gi).wait()
            _cidx(gi, 0).wait()
            _cidx(gi, 1).wait()
            for t in range(16):
                rowb[gi % 3, pl.ds(t * 16, 16)] = (
                    rowb[gi % 3, pl.ds(t * 16, 16)] + chunk * N)

        def _cidx(grp, half):
            return pltpu.make_async_copy(
                colp_hbm.at[pl.ds(ebase + grp * 256 + half * 128, 128)],
                colb.at[grp % 3, half], csem.at[grp % 3])

        def _gather(j):
            jo = j // 2
            off = pl.multiple_of((j % 2) * 128, 128)
            return pltpu.make_async_copy(
                g_hbm.at[rowb.at[jo % 3, pl.ds(off, 128)]],
                dbuf.at[j % 3], gsem.at[j % 3])

        def _scatter(j):
            return pltpu.make_async_copy(
                dbuf.at[j % 3], shared.at[colb.at[(j // 2) % 3, j % 2]],
                ssem.at[j % 3])

        _ridx(0).start()
        _cidx(0, 0).start()
        _cidx(0, 1).start()
        _idx_ready(0)
        _ridx(1).start()
        _cidx(1, 0).start()
        _cidx(1, 1).start()
        _gather(0).start()
        _gather(1).start()

        def _step(j, _):
            _gather(j).wait()
            pltpu.async_copy(
                dbuf.at[j % 3],
                shared.at[colb.at[(j // 2) % 3, j % 2]],
                ssem.at[j % 3], add=True)

            @pl.when(j >= 1)
            def _():
                _scatter(j - 1).wait()

            @pl.when(j + 2 < nst)
            def _():
                @pl.when((j + 2) % 2 == 0)
                def _():
                    gi = (j + 2) // 2
                    _idx_ready(gi)

                    @pl.when(gi + 1 < ngrp)
                    def _():
                        _ridx(gi + 1).start()
                        _cidx(gi + 1, 0).start()
                        _cidx(gi + 1, 1).start()
                _gather(j + 2).start()
            return 0

        lax.fori_loop(0, nst, _step, 0)
        _scatter(nst - 1).wait()
        plsc.subcore_barrier()
        pltpu.sync_copy(shared.at[pl.ds(s * 624, 624)],
                        s_hbm.at[chunk, pl.ds(s * 624, 624)])

        @pl.when(s == 15)
        def _():
            pltpu.sync_copy(shared.at[pl.ds(9984, 16)],
                            s_hbm.at[chunk, pl.ds(9984, 16)])
        if p == 0:
            plsc.subcore_barrier()


def _k1_body(x_ref, w0_ref, b0_ref, deg_ref, g_ref):
    h = jnp.dot(x_ref[...], w0_ref[...], preferred_element_type=jnp.float32)
    h = jnp.maximum(h + b0_ref[...], 0.0)
    dinv = lax.rsqrt(deg_ref[...] + 1.0)
    g = h * dinv
    for cc in range(NCH):
        g_ref[cc] = g[:, cc * CW:(cc + 1) * CW]


def _k1_call(x, W0, b0r, deg1):
    return pl.pallas_call(
        _k1_body,
        grid=(N // RB,),
        in_specs=[
            pl.BlockSpec((RB, FIN), lambda i: (i, 0)),
            pl.BlockSpec((FIN, FHID), lambda i: (0, 0)),
            pl.BlockSpec((1, FHID), lambda i: (0, 0)),
            pl.BlockSpec((RB, 1), lambda i: (i, 0)),
        ],
        out_specs=pl.BlockSpec((NCH, RB, CW), lambda i: (0, i, 0)),
        out_shape=jax.ShapeDtypeStruct((NCH, N, CW), jnp.float32),
        compiler_params=pltpu.CompilerParams(
            dimension_semantics=("arbitrary",)),
    )(x, W0, b0r, deg1)


def _k2_body(s_ref, deg_ref, w1_ref, b1_ref, gam_ref, bet_ref,
             mu_ref, var_ref, o_ref):
    dinv = lax.rsqrt(deg_ref[...] + 1.0)
    ii = lax.broadcasted_iota(jnp.int32, (CW, CW), 0)
    jj = lax.broadcasted_iota(jnp.int32, (CW, CW), 1)
    a_op = (jnp.where(ii == jj, 2.0, 0.0)
            + jnp.where((ii // 2) == (jj // 2), 0.5, 0.0)
            + jnp.where((ii // 4) == (jj // 4), 0.25, 0.0))
    acc = jnp.zeros((RB, FOUT), jnp.float32)
    for cc in range(NCH):
        t = s_ref[cc] * dinv
        wc = jnp.dot(a_op, w1_ref[cc], preferred_element_type=jnp.float32)
        acc = acc + jnp.dot(t, wc, preferred_element_type=jnp.float32)
    scale = gam_ref[...] * lax.rsqrt(var_ref[...] + 1e-5)
    o_ref[...] = acc * scale + (b1_ref[...] - mu_ref[...]) * scale + bet_ref[...]


def _k2_call(S, deg1, W1r, b1r, gamr, betr, mur, varr):
    vec = pl.BlockSpec((1, FOUT), lambda i: (0, 0))
    return pl.pallas_call(
        _k2_body,
        grid=(N // RB,),
        in_specs=[
            pl.BlockSpec((NCH, RB, CW), lambda i: (0, i, 0)),
            pl.BlockSpec((RB, 1), lambda i: (i, 0)),
            pl.BlockSpec((NCH, CW, FOUT), lambda i: (0, 0, 0)),
            vec, vec, vec, vec, vec,
        ],
        out_specs=pl.BlockSpec((RB, FOUT), lambda i: (i, 0)),
        out_shape=jax.ShapeDtypeStruct((N, FOUT), jnp.float32),
        compiler_params=pltpu.CompilerParams(
            dimension_semantics=("arbitrary",)),
    )(S, deg1, W1r, b1r, gamr, betr, mur, varr)


def kernel(x, edge_index, W0, b0, W1, b1, bn_gamma, bn_beta, bn_mean, bn_var):
    rowp = edge_index[0]
    colp = edge_index[1]

    degp = _deg_kernel(colp)
    deg1 = (degp[:N] + degp[NSP:NSP + N]).reshape(N, 1)

    g4 = _k1_call(x, W0, b0.reshape(1, FHID), deg1)
    g_flat = g4.reshape(NCH * N, CW)

    S = _prop_kernel(g_flat, rowp, colp)


    return _k2_call(S, deg1, W1.reshape(NCH, CW, FOUT),
                    b1.reshape(1, FOUT), bn_gamma.reshape(1, FOUT),
                    bn_beta.reshape(1, FOUT), bn_mean.reshape(1, FOUT),
                    bn_var.reshape(1, FOUT))
